# baseline (speedup 1.0000x reference)
import functools

import jax
import jax.numpy as jnp
from jax import lax
from jax.experimental import pallas as pl
from jax.experimental.pallas import tpu as pltpu
from jax.experimental.pallas import tpu_sc as plsc

N = 10000
DIN = 128
H = 128
OUT = 138
G = 64

NP = 10240
BLK = 512
NBLK = NP // BLK
OUTP = 256

NW = 32
CH = 128
NCH = 80
EP = NW * NCH * CH
STRIPE = NP // 16

NCHT = 2 * NCH
SPLIT0 = 96
SPLIT1 = NCHT - SPLIT0
NPHASE = 4
IR = SPLIT1 // NPHASE if SPLIT1 >= SPLIT0 else SPLIT0 // NPHASE

_HIGH = lax.Precision.HIGHEST


@functools.cache
def _build_sc_degree():
    mesh = plsc.VectorSubcoreMesh(core_axis_name="c", subcore_axis_name="s")

    @functools.partial(
        pl.kernel,
        out_type=jax.ShapeDtypeStruct((NW * NP,), jnp.float32),
        mesh=mesh,
        scratch_types=[
            pltpu.VMEM((NP,), jnp.float32),
            pltpu.VMEM((NCH, CH), jnp.int32),
        ],
        compiler_params=pltpu.CompilerParams(needs_layout_passes=False),
    )
    def deg_kernel(dst_hbm, zero_hbm, out_hbm, hist, dstv):
        cid = lax.axis_index("c")
        sid = lax.axis_index("s")
        wid = sid * 2 + cid
        pltpu.sync_copy(zero_hbm, hist)
        pltpu.sync_copy(dst_hbm.at[pl.ds(wid * NCH, NCH)], dstv)
        ones16 = jnp.ones((16,), jnp.float32)

        @pl.loop(0, NCH)
        def _(j):
            for k in range(8):
                idx = dstv[j, pl.ds(k * 16, 16)]
                plsc.addupdate_scatter(hist, [idx], ones16)

        pltpu.sync_copy(hist, out_hbm.at[pl.ds(wid * NP, NP)])

    return deg_kernel


def _sc_degree(dst2d, zflat):
    return _build_sc_degree()(dst2d, zflat)


@functools.cache
def _build_sc_propagate():
    mesh = plsc.VectorSubcoreMesh(core_axis_name="c", subcore_axis_name="s")

    @functools.partial(
        pl.kernel,
        out_type=jax.ShapeDtypeStruct((2, NP, H), jnp.float32),
        mesh=mesh,
        scratch_types=[
            pltpu.VMEM_SHARED((NP, H), jnp.float32),
            pltpu.VMEM((IR, CH), jnp.int32),
            pltpu.VMEM((IR, CH), jnp.int32),
            pltpu.VMEM((CH, H), jnp.float32),
            pltpu.VMEM((CH, H), jnp.float32),
            pltpu.SemaphoreType.DMA,
            pltpu.SemaphoreType.DMA,
        ],
    )
    def prop_kernel(y_hbm, src_hbm, dst_hbm, zero_hbm, out_hbm,
                    acc, srcv, dstv, ra, rb, sa, sb):
        cid = lax.axis_index("c")
        sid = lax.axis_index("s")

        pltpu.sync_copy(zero_hbm, ra)
        for k in range(STRIPE // CH):
            pltpu.sync_copy(ra, acc.at[pl.ds(sid * STRIPE + k * CH, CH)])

        plsc.subcore_barrier()

        def run_edges(base_row, nch):
            hn = nch // NPHASE
            for p in range(NPHASE):
                row0 = base_row + p * hn
                pltpu.sync_copy(src_hbm.at[pl.ds(row0, hn)],
                                srcv.at[pl.ds(0, hn)])
                pltpu.sync_copy(dst_hbm.at[pl.ds(row0, hn)],
                                dstv.at[pl.ds(0, hn)])

                pltpu.async_copy(y_hbm.at[srcv.at[0]], ra, sa)

                @pl.loop(0, hn, step=2)
                def _(j):
                    pltpu.async_copy(y_hbm.at[srcv.at[j + 1]], rb, sb)
                    pltpu.make_async_copy(y_hbm.at[srcv.at[j]], ra, sa).wait()
                    pltpu.sync_copy(ra, acc.at[dstv.at[j]], add=True)

                    @pl.when(j + 2 < hn)
                    def _():
                        pltpu.async_copy(y_hbm.at[srcv.at[j + 2]], ra, sa)

                    pltpu.make_async_copy(y_hbm.at[srcv.at[j + 1]], rb, sb).wait()
                    pltpu.sync_copy(rb, acc.at[dstv.at[j + 1]], add=True)

        @pl.when(cid == 0)
        def _():
            run_edges(sid * SPLIT0, SPLIT0)

        @pl.when(cid == 1)
        def _():
            run_edges(16 * SPLIT0 + sid * SPLIT1, SPLIT1)

        plsc.subcore_barrier()
        pltpu.sync_copy(acc.at[pl.ds(sid * STRIPE, STRIPE)],
                        out_hbm.at[cid, pl.ds(sid * STRIPE, STRIPE)])

    return prop_kernel


def _sc_propagate(y, src2d, dst2d, zero128):
    return _build_sc_propagate()(y, src2d, dst2d, zero128)


def _mm_body(x_ref, w_ref, o_ref):
    o_ref[...] = jnp.dot(x_ref[...], w_ref[...],
                         preferred_element_type=jnp.float32, precision=_HIGH)


def _tc_matmul(xp, w1):
    return pl.pallas_call(
        _mm_body,
        grid=(NBLK,),
        in_specs=[
            pl.BlockSpec((BLK, DIN), lambda i: (i, 0)),
            pl.BlockSpec((DIN, H), lambda i: (0, 0)),
        ],
        out_specs=pl.BlockSpec((BLK, H), lambda i: (i, 0)),
        out_shape=jax.ShapeDtypeStruct((NP, H), jnp.float32),
    )(xp, w1)


def _first_body(deg_ref, xw_ref, y_ref, dinv_ref):
    s = jnp.sum(deg_ref[:, 0], axis=0)
    dinv = lax.rsqrt(s + 1.0)
    r = lax.broadcasted_iota(jnp.int32, (BLK, H), 0)
    c = lax.broadcasted_iota(jnp.int32, (BLK, H), 1)
    esel = (lax.broadcasted_iota(jnp.int32, (BLK, BLK // 128), 1)
            == r[:, :BLK // 128] // 128).astype(jnp.float32)
    expand = lax.dot_general(esel, dinv, (((1,), (0,)), ((), ())),
                             preferred_element_type=jnp.float32,
                             precision=_HIGH)
    onehot = jnp.where(c == r % 128, expand, 0.0)
    dinvb = lax.dot_general(onehot, jnp.ones((H, H), jnp.float32),
                            (((1,), (0,)), ((), ())),
                            preferred_element_type=jnp.float32,
                            precision=_HIGH)
    y_ref[...] = xw_ref[...] * dinvb
    dinv_ref[...] = dinvb


def _tc_first(degp, xw1):
    return pl.pallas_call(
        _first_body,
        grid=(NBLK,),
        in_specs=[
            pl.BlockSpec((NW, 1, BLK // 128, 128), lambda i: (0, i, 0, 0)),
            pl.BlockSpec((BLK, H), lambda i: (i, 0)),
        ],
        out_specs=[
            pl.BlockSpec((BLK, H), lambda i: (i, 0)),
            pl.BlockSpec((BLK, H), lambda i: (i, 0)),
        ],
        out_shape=[
            jax.ShapeDtypeStruct((NP, H), jnp.float32),
            jax.ShapeDtypeStruct((NP, H), jnp.float32),
        ],
    )(degp, xw1)


def _mid_body(p_ref, y_ref, dinv_ref, b_ref, w_ref, o_ref):
    h = jnp.maximum(
        (p_ref[0] + p_ref[1] + y_ref[...]) * dinv_ref[...] + b_ref[...], 0.0)
    o_ref[...] = jnp.dot(h, w_ref[...],
                         preferred_element_type=jnp.float32,
                         precision=_HIGH) * dinv_ref[...]


def _tc_mid(p1, y1, dinvb, b1r, w2):
    return pl.pallas_call(
        _mid_body,
        grid=(NBLK,),
        in_specs=[
            pl.BlockSpec((2, BLK, H), lambda i: (0, i, 0)),
            pl.BlockSpec((BLK, H), lambda i: (i, 0)),
            pl.BlockSpec((BLK, H), lambda i: (i, 0)),
            pl.BlockSpec((1, H), lambda i: (0, 0)),
            pl.BlockSpec((H, H), lambda i: (0, 0)),
        ],
        out_specs=pl.BlockSpec((BLK, H), lambda i: (i, 0)),
        out_shape=jax.ShapeDtypeStruct((NP, H), jnp.float32),
    )(p1, y1, dinvb, b1r, w2)


def _final_body(p_ref, y_ref, dinv_ref, b_ref, batch_ref, wl_ref, bl_ref,
                o_ref, acc, cnt):
    i = pl.program_id(0)
    h = jnp.maximum(
        (p_ref[0] + p_ref[1] + y_ref[...]) * dinv_ref[...] + b_ref[...], 0.0)
    bid = batch_ref[0]
    gids = lax.broadcasted_iota(jnp.int32, (G, BLK), 0)
    maskT = (jnp.broadcast_to(bid, (G, BLK)) == gids).astype(jnp.float32)
    contrib = lax.dot_general(maskT, h, (((1,), (0,)), ((), ())),
                              preferred_element_type=jnp.float32,
                              precision=_HIGH)
    ones = jnp.ones((BLK, H), jnp.float32)
    ccnt = lax.dot_general(maskT, ones, (((1,), (0,)), ((), ())),
                           preferred_element_type=jnp.float32,
                           precision=_HIGH)

    @pl.when(i == 0)
    def _():
        acc[...] = contrib
        cnt[...] = ccnt

    @pl.when(i > 0)
    def _():
        acc[...] = acc[...] + contrib
        cnt[...] = cnt[...] + ccnt

    @pl.when(i == NBLK - 1)
    def _():
        g = acc[...] / jnp.maximum(cnt[...], 1.0)
        o_ref[...] = jnp.dot(g, wl_ref[...],
                             preferred_element_type=jnp.float32,
                             precision=_HIGH) + bl_ref[...]


def _tc_final(p2, y2, dinvb, b2r, batchr, wlinp, blinp):
    return pl.pallas_call(
        _final_body,
        grid=(NBLK,),
        in_specs=[
            pl.BlockSpec((2, BLK, H), lambda i: (0, i, 0)),
            pl.BlockSpec((BLK, H), lambda i: (i, 0)),
            pl.BlockSpec((BLK, H), lambda i: (i, 0)),
            pl.BlockSpec((1, H), lambda i: (0, 0)),
            pl.BlockSpec((1, 1, BLK), lambda i: (i, 0, 0)),
            pl.BlockSpec((H, OUTP), lambda i: (0, 0)),
            pl.BlockSpec((1, OUTP), lambda i: (0, 0)),
        ],
        out_specs=pl.BlockSpec((G, OUTP), lambda i: (0, 0)),
        out_shape=jax.ShapeDtypeStruct((G, OUTP), jnp.float32),
        scratch_shapes=[
            pltpu.VMEM((G, H), jnp.float32),
            pltpu.VMEM((G, H), jnp.float32),
        ],
    )(p2, y2, dinvb, b2r, batchr, wlinp, blinp)


def kernel(x, edge_index, batch, W1, b1, W2, b2, Wlin, blin):
    E = edge_index.shape[1]
    pad_e = EP - E
    src2d = jnp.concatenate(
        [edge_index[0], jnp.full((pad_e,), N, jnp.int32)]).reshape(NW * NCH, CH)
    dst2d = jnp.concatenate(
        [edge_index[1], jnp.full((pad_e,), N, jnp.int32)]).reshape(NW * NCH, CH)
    xp = jnp.zeros((NP, DIN), jnp.float32).at[:N].set(x)
    batchr = jnp.concatenate(
        [batch, jnp.full((NP - N,), G, jnp.int32)]).reshape(NBLK, 1, BLK)
    b1r = b1.reshape(1, H)
    b2r = b2.reshape(1, H)
    wlinp = jnp.zeros((H, OUTP), jnp.float32).at[:, :OUT].set(Wlin)
    blinp = jnp.zeros((1, OUTP), jnp.float32).at[0, :OUT].set(blin)

    zero128 = jnp.zeros((CH, H), jnp.float32)
    zflat = jnp.zeros((NP,), jnp.float32)

    degp = _sc_degree(dst2d, zflat).reshape(NW, NBLK, BLK // 128, 128)
    xw1 = _tc_matmul(xp, W1)
    y1, dinvb = _tc_first(degp, xw1)
    p1 = _sc_propagate(y1, src2d, dst2d, zero128)
    y2 = _tc_mid(p1, y1, dinvb, b1r, W2)
    p2 = _sc_propagate(y2, src2d, dst2d, zero128)
    logits_p = _tc_final(p2, y2, dinvb, b2r, batchr, wlinp, blinp)
    return logits_p[:, :OUT]


# speedup vs baseline: 2.3207x; 2.3207x over previous
import functools

import jax
import jax.numpy as jnp
from jax import lax
from jax.experimental import pallas as pl
from jax.experimental.pallas import tpu as pltpu
from jax.experimental.pallas import tpu_sc as plsc

N = 10000
DIN = 128
H = 128
OUT = 138
G = 64

NP = 10240
BLK = 512
NBLK = NP // BLK
OUTP = 256

NW = 32
CH = 128
NCH = 80
EP = NW * NCH * CH
STRIPE = NP // 16

NCHT = 2 * NCH
SPLIT0 = 128
SPLIT1 = NCHT - SPLIT0
NPHASE = 4
IR = SPLIT1 // NPHASE if SPLIT1 >= SPLIT0 else SPLIT0 // NPHASE

_HIGH = lax.Precision.HIGHEST


@functools.cache
def _build_sc_degree():
    mesh = plsc.VectorSubcoreMesh(core_axis_name="c", subcore_axis_name="s")

    @functools.partial(
        pl.kernel,
        out_type=jax.ShapeDtypeStruct((NW * NP,), jnp.float32),
        mesh=mesh,
        scratch_types=[
            pltpu.VMEM((NP,), jnp.float32),
            pltpu.VMEM((NCH, CH), jnp.int32),
        ],
        compiler_params=pltpu.CompilerParams(needs_layout_passes=False),
    )
    def deg_kernel(dst_hbm, zero_hbm, out_hbm, hist, dstv):
        cid = lax.axis_index("c")
        sid = lax.axis_index("s")
        wid = sid * 2 + cid
        pltpu.sync_copy(zero_hbm, hist)
        pltpu.sync_copy(dst_hbm.at[pl.ds(wid * NCH, NCH)], dstv)
        ones16 = jnp.ones((16,), jnp.float32)

        @pl.loop(0, NCH)
        def _(j):
            for k in range(8):
                idx = dstv[j, pl.ds(k * 16, 16)]
                plsc.addupdate_scatter(hist, [idx], ones16)

        pltpu.sync_copy(hist, out_hbm.at[pl.ds(wid * NP, NP)])

    return deg_kernel


def _sc_degree(dst2d, zflat):
    return _build_sc_degree()(dst2d, zflat)


@functools.cache
def _build_sc_propagate():
    mesh = plsc.VectorSubcoreMesh(core_axis_name="c", subcore_axis_name="s")

    @functools.partial(
        pl.kernel,
        out_type=jax.ShapeDtypeStruct((2, NP, H), jnp.float32),
        mesh=mesh,
        scratch_types=[
            pltpu.VMEM_SHARED((NP, H), jnp.float32),
            pltpu.VMEM((IR, CH), jnp.int32),
            pltpu.VMEM((IR, CH), jnp.int32),
            pltpu.VMEM((CH, H), jnp.float32),
            pltpu.VMEM((CH, H), jnp.float32),
            pltpu.SemaphoreType.DMA,
            pltpu.SemaphoreType.DMA,
        ],
    )
    def prop_kernel(y_hbm, src_hbm, dst_hbm, zero_hbm, out_hbm,
                    acc, srcv, dstv, ra, rb, sa, sb):
        cid = lax.axis_index("c")
        sid = lax.axis_index("s")

        pltpu.sync_copy(zero_hbm, ra)
        for k in range(STRIPE // CH):
            pltpu.sync_copy(ra, acc.at[pl.ds(sid * STRIPE + k * CH, CH)])

        plsc.subcore_barrier()

        def run_edges(base_row, nch):
            hn = nch // NPHASE
            for p in range(NPHASE):
                row0 = base_row + p * hn
                pltpu.sync_copy(src_hbm.at[pl.ds(row0, hn)],
                                srcv.at[pl.ds(0, hn)])
                pltpu.sync_copy(dst_hbm.at[pl.ds(row0, hn)],
                                dstv.at[pl.ds(0, hn)])

                pltpu.async_copy(y_hbm.at[srcv.at[0]], ra, sa)

                @pl.loop(0, hn, step=2)
                def _(j):
                    pltpu.async_copy(y_hbm.at[srcv.at[j + 1]], rb, sb)
                    pltpu.make_async_copy(y_hbm.at[srcv.at[j]], ra, sa).wait()
                    pltpu.sync_copy(ra, acc.at[dstv.at[j]], add=True)

                    @pl.when(j + 2 < hn)
                    def _():
                        pltpu.async_copy(y_hbm.at[srcv.at[j + 2]], ra, sa)

                    pltpu.make_async_copy(y_hbm.at[srcv.at[j + 1]], rb, sb).wait()
                    pltpu.sync_copy(rb, acc.at[dstv.at[j + 1]], add=True)

        @pl.when(cid == 0)
        def _():
            run_edges(sid * SPLIT0, SPLIT0)

        @pl.when(cid == 1)
        def _():
            run_edges(16 * SPLIT0 + sid * SPLIT1, SPLIT1)

        plsc.subcore_barrier()
        pltpu.sync_copy(acc.at[pl.ds(sid * STRIPE, STRIPE)],
                        out_hbm.at[cid, pl.ds(sid * STRIPE, STRIPE)])

    return prop_kernel


def _sc_propagate(y, src2d, dst2d, zero128):
    return _build_sc_propagate()(y, src2d, dst2d, zero128)


def _mm_body(x_ref, w_ref, o_ref):
    o_ref[...] = jnp.dot(x_ref[...], w_ref[...],
                         preferred_element_type=jnp.float32, precision=_HIGH)


def _tc_matmul(xp, w1):
    return pl.pallas_call(
        _mm_body,
        grid=(NBLK,),
        in_specs=[
            pl.BlockSpec((BLK, DIN), lambda i: (i, 0)),
            pl.BlockSpec((DIN, H), lambda i: (0, 0)),
        ],
        out_specs=pl.BlockSpec((BLK, H), lambda i: (i, 0)),
        out_shape=jax.ShapeDtypeStruct((NP, H), jnp.float32),
    )(xp, w1)


def _first_body(deg_ref, xw_ref, y_ref, dinv_ref):
    s = jnp.sum(deg_ref[:, 0], axis=0)
    dinv = lax.rsqrt(s + 1.0)
    r = lax.broadcasted_iota(jnp.int32, (BLK, H), 0)
    c = lax.broadcasted_iota(jnp.int32, (BLK, H), 1)
    esel = (lax.broadcasted_iota(jnp.int32, (BLK, BLK // 128), 1)
            == r[:, :BLK // 128] // 128).astype(jnp.float32)
    expand = lax.dot_general(esel, dinv, (((1,), (0,)), ((), ())),
                             preferred_element_type=jnp.float32,
                             precision=_HIGH)
    onehot = jnp.where(c == r % 128, expand, 0.0)
    dinvb = lax.dot_general(onehot, jnp.ones((H, H), jnp.float32),
                            (((1,), (0,)), ((), ())),
                            preferred_element_type=jnp.float32,
                            precision=_HIGH)
    y_ref[...] = xw_ref[...] * dinvb
    dinv_ref[...] = dinvb


def _tc_first(degp, xw1):
    return pl.pallas_call(
        _first_body,
        grid=(NBLK,),
        in_specs=[
            pl.BlockSpec((NW, 1, BLK // 128, 128), lambda i: (0, i, 0, 0)),
            pl.BlockSpec((BLK, H), lambda i: (i, 0)),
        ],
        out_specs=[
            pl.BlockSpec((BLK, H), lambda i: (i, 0)),
            pl.BlockSpec((BLK, H), lambda i: (i, 0)),
        ],
        out_shape=[
            jax.ShapeDtypeStruct((NP, H), jnp.float32),
            jax.ShapeDtypeStruct((NP, H), jnp.float32),
        ],
    )(degp, xw1)


def _mid_body(p_ref, y_ref, dinv_ref, b_ref, w_ref, o_ref):
    h = jnp.maximum(
        (p_ref[0] + p_ref[1] + y_ref[...]) * dinv_ref[...] + b_ref[...], 0.0)
    o_ref[...] = jnp.dot(h, w_ref[...],
                         preferred_element_type=jnp.float32,
                         precision=_HIGH) * dinv_ref[...]


def _tc_mid(p1, y1, dinvb, b1r, w2):
    return pl.pallas_call(
        _mid_body,
        grid=(NBLK,),
        in_specs=[
            pl.BlockSpec((2, BLK, H), lambda i: (0, i, 0)),
            pl.BlockSpec((BLK, H), lambda i: (i, 0)),
            pl.BlockSpec((BLK, H), lambda i: (i, 0)),
            pl.BlockSpec((1, H), lambda i: (0, 0)),
            pl.BlockSpec((H, H), lambda i: (0, 0)),
        ],
        out_specs=pl.BlockSpec((BLK, H), lambda i: (i, 0)),
        out_shape=jax.ShapeDtypeStruct((NP, H), jnp.float32),
    )(p1, y1, dinvb, b1r, w2)


def _final_body(p_ref, y_ref, dinv_ref, b_ref, batch_ref, wl_ref, bl_ref,
                o_ref, acc, cnt):
    i = pl.program_id(0)
    h = jnp.maximum(
        (p_ref[0] + p_ref[1] + y_ref[...]) * dinv_ref[...] + b_ref[...], 0.0)
    bid = batch_ref[0]
    gids = lax.broadcasted_iota(jnp.int32, (G, BLK), 0)
    maskT = (jnp.broadcast_to(bid, (G, BLK)) == gids).astype(jnp.float32)
    contrib = lax.dot_general(maskT, h, (((1,), (0,)), ((), ())),
                              preferred_element_type=jnp.float32,
                              precision=_HIGH)
    ones = jnp.ones((BLK, H), jnp.float32)
    ccnt = lax.dot_general(maskT, ones, (((1,), (0,)), ((), ())),
                           preferred_element_type=jnp.float32,
                           precision=_HIGH)

    @pl.when(i == 0)
    def _():
        acc[...] = contrib
        cnt[...] = ccnt

    @pl.when(i > 0)
    def _():
        acc[...] = acc[...] + contrib
        cnt[...] = cnt[...] + ccnt

    @pl.when(i == NBLK - 1)
    def _():
        g = acc[...] / jnp.maximum(cnt[...], 1.0)
        o_ref[...] = jnp.dot(g, wl_ref[...],
                             preferred_element_type=jnp.float32,
                             precision=_HIGH) + bl_ref[...]


def _tc_final(p2, y2, dinvb, b2r, batchr, wlinp, blinp):
    return pl.pallas_call(
        _final_body,
        grid=(NBLK,),
        in_specs=[
            pl.BlockSpec((2, BLK, H), lambda i: (0, i, 0)),
            pl.BlockSpec((BLK, H), lambda i: (i, 0)),
            pl.BlockSpec((BLK, H), lambda i: (i, 0)),
            pl.BlockSpec((1, H), lambda i: (0, 0)),
            pl.BlockSpec((1, 1, BLK), lambda i: (i, 0, 0)),
            pl.BlockSpec((H, OUTP), lambda i: (0, 0)),
            pl.BlockSpec((1, OUTP), lambda i: (0, 0)),
        ],
        out_specs=pl.BlockSpec((G, OUTP), lambda i: (0, 0)),
        out_shape=jax.ShapeDtypeStruct((G, OUTP), jnp.float32),
        scratch_shapes=[
            pltpu.VMEM((G, H), jnp.float32),
            pltpu.VMEM((G, H), jnp.float32),
        ],
    )(p2, y2, dinvb, b2r, batchr, wlinp, blinp)


def kernel(x, edge_index, batch, W1, b1, W2, b2, Wlin, blin):
    E = edge_index.shape[1]
    pad_e = EP - E
    pad_idx = N + jnp.arange(pad_e, dtype=jnp.int32) % (NP - N)
    src2d = jnp.concatenate([edge_index[0], pad_idx]).reshape(NW * NCH, CH)
    dst2d = jnp.concatenate([edge_index[1], pad_idx]).reshape(NW * NCH, CH)
    xp = jnp.zeros((NP, DIN), jnp.float32).at[:N].set(x)
    batchr = jnp.concatenate(
        [batch, jnp.full((NP - N,), G, jnp.int32)]).reshape(NBLK, 1, BLK)
    b1r = b1.reshape(1, H)
    b2r = b2.reshape(1, H)
    wlinp = jnp.zeros((H, OUTP), jnp.float32).at[:, :OUT].set(Wlin)
    blinp = jnp.zeros((1, OUTP), jnp.float32).at[0, :OUT].set(blin)

    zero128 = jnp.zeros((CH, H), jnp.float32)
    zflat = jnp.zeros((NP,), jnp.float32)

    degp = _sc_degree(dst2d, zflat).reshape(NW, NBLK, BLK // 128, 128)
    xw1 = _tc_matmul(xp, W1)
    y1, dinvb = _tc_first(degp, xw1)
    p1 = _sc_propagate(y1, src2d, dst2d, zero128)
    y2 = _tc_mid(p1, y1, dinvb, b1r, W2)
    p2 = _sc_propagate(y2, src2d, dst2d, zero128)
    logits_p = _tc_final(p2, y2, dinvb, b2r, batchr, wlinp, blinp)
    return logits_p[:, :OUT]


# speedup vs baseline: 2.7267x; 1.1750x over previous
import functools

import jax
import jax.numpy as jnp
from jax import lax
from jax.experimental import pallas as pl
from jax.experimental.pallas import tpu as pltpu
from jax.experimental.pallas import tpu_sc as plsc

N = 10000
DIN = 128
H = 128
OUT = 138
G = 64

NP = 10240
BLK = 512
NBLK = NP // BLK
OUTP = 256

NW = 32
CH = 128
NCH = 80
EP = NW * NCH * CH
STRIPE = NP // 16

NCHT = 2 * NCH
SPLIT0 = 96
SPLIT1 = NCHT - SPLIT0
NPHASE = 4
IR = SPLIT1 // NPHASE if SPLIT1 >= SPLIT0 else SPLIT0 // NPHASE

_HIGH = lax.Precision.HIGHEST


@functools.cache
def _build_sc_degree():
    mesh = plsc.VectorSubcoreMesh(core_axis_name="c", subcore_axis_name="s")

    @functools.partial(
        pl.kernel,
        out_type=jax.ShapeDtypeStruct((NW * NP,), jnp.float32),
        mesh=mesh,
        scratch_types=[
            pltpu.VMEM((NP,), jnp.float32),
            pltpu.VMEM((NCH, CH), jnp.int32),
        ],
        compiler_params=pltpu.CompilerParams(needs_layout_passes=False),
    )
    def deg_kernel(dst_hbm, zero_hbm, out_hbm, hist, dstv):
        cid = lax.axis_index("c")
        sid = lax.axis_index("s")
        wid = sid * 2 + cid
        pltpu.sync_copy(zero_hbm, hist)
        pltpu.sync_copy(dst_hbm.at[pl.ds(wid * NCH, NCH)], dstv)
        ones16 = jnp.ones((16,), jnp.float32)

        @pl.loop(0, NCH)
        def _(j):
            for k in range(8):
                idx = dstv[j, pl.ds(k * 16, 16)]
                plsc.addupdate_scatter(hist, [idx], ones16)

        pltpu.sync_copy(hist, out_hbm.at[pl.ds(wid * NP, NP)])

    return deg_kernel


def _sc_degree(dst2d, zflat):
    return _build_sc_degree()(dst2d, zflat)


@functools.cache
def _build_sc_propagate():
    mesh = plsc.VectorSubcoreMesh(core_axis_name="c", subcore_axis_name="s")

    @functools.partial(
        pl.kernel,
        out_type=jax.ShapeDtypeStruct((2, NP, H), jnp.float32),
        mesh=mesh,
        scratch_types=[
            pltpu.VMEM_SHARED((NP, H), jnp.float32),
            pltpu.VMEM((IR, CH), jnp.int32),
            pltpu.VMEM((IR, CH), jnp.int32),
            pltpu.VMEM((CH, H), jnp.float32),
            pltpu.VMEM((CH, H), jnp.float32),
            pltpu.SemaphoreType.DMA,
            pltpu.SemaphoreType.DMA,
        ],
    )
    def prop_kernel(y_hbm, src_hbm, dst_hbm, zero_hbm, out_hbm,
                    acc, srcv, dstv, ra, rb, sa, sb):
        cid = lax.axis_index("c")
        sid = lax.axis_index("s")

        pltpu.sync_copy(zero_hbm, ra)
        for k in range(STRIPE // CH):
            pltpu.sync_copy(ra, acc.at[pl.ds(sid * STRIPE + k * CH, CH)])

        plsc.subcore_barrier()

        def run_edges(base_row, nch):
            hn = nch // NPHASE
            for p in range(NPHASE):
                row0 = base_row + p * hn
                pltpu.sync_copy(src_hbm.at[pl.ds(row0, hn)],
                                srcv.at[pl.ds(0, hn)])
                pltpu.sync_copy(dst_hbm.at[pl.ds(row0, hn)],
                                dstv.at[pl.ds(0, hn)])

                pltpu.async_copy(y_hbm.at[srcv.at[0]], ra, sa)

                @pl.loop(0, hn, step=2)
                def _(j):
                    pltpu.async_copy(y_hbm.at[srcv.at[j + 1]], rb, sb)
                    pltpu.make_async_copy(y_hbm.at[srcv.at[j]], ra, sa).wait()
                    pltpu.sync_copy(ra, acc.at[dstv.at[j]], add=True)

                    @pl.when(j + 2 < hn)
                    def _():
                        pltpu.async_copy(y_hbm.at[srcv.at[j + 2]], ra, sa)

                    pltpu.make_async_copy(y_hbm.at[srcv.at[j + 1]], rb, sb).wait()
                    pltpu.sync_copy(rb, acc.at[dstv.at[j + 1]], add=True)

        @pl.when(cid == 0)
        def _():
            run_edges(sid * SPLIT0, SPLIT0)

        @pl.when(cid == 1)
        def _():
            run_edges(16 * SPLIT0 + sid * SPLIT1, SPLIT1)

        plsc.subcore_barrier()
        pltpu.sync_copy(acc.at[pl.ds(sid * STRIPE, STRIPE)],
                        out_hbm.at[cid, pl.ds(sid * STRIPE, STRIPE)])

    return prop_kernel


def _sc_propagate(y, src2d, dst2d, zero128):
    return _build_sc_propagate()(y, src2d, dst2d, zero128)


def _mm_body(x_ref, w_ref, o_ref):
    o_ref[...] = jnp.dot(x_ref[...], w_ref[...],
                         preferred_element_type=jnp.float32, precision=_HIGH)


def _tc_matmul(xp, w1):
    return pl.pallas_call(
        _mm_body,
        grid=(NBLK,),
        in_specs=[
            pl.BlockSpec((BLK, DIN), lambda i: (i, 0)),
            pl.BlockSpec((DIN, H), lambda i: (0, 0)),
        ],
        out_specs=pl.BlockSpec((BLK, H), lambda i: (i, 0)),
        out_shape=jax.ShapeDtypeStruct((NP, H), jnp.float32),
    )(xp, w1)


def _first_body(deg_ref, xw_ref, y_ref, dinv_ref):
    s = jnp.sum(deg_ref[:, 0], axis=0)
    dinv = lax.rsqrt(s + 1.0)
    r = lax.broadcasted_iota(jnp.int32, (BLK, H), 0)
    c = lax.broadcasted_iota(jnp.int32, (BLK, H), 1)
    esel = (lax.broadcasted_iota(jnp.int32, (BLK, BLK // 128), 1)
            == r[:, :BLK // 128] // 128).astype(jnp.float32)
    expand = lax.dot_general(esel, dinv, (((1,), (0,)), ((), ())),
                             preferred_element_type=jnp.float32,
                             precision=_HIGH)
    onehot = jnp.where(c == r % 128, expand, 0.0)
    dinvb = lax.dot_general(onehot, jnp.ones((H, H), jnp.float32),
                            (((1,), (0,)), ((), ())),
                            preferred_element_type=jnp.float32,
                            precision=_HIGH)
    y_ref[...] = xw_ref[...] * dinvb
    dinv_ref[...] = dinvb


def _tc_first(degp, xw1):
    return pl.pallas_call(
        _first_body,
        grid=(NBLK,),
        in_specs=[
            pl.BlockSpec((NW, 1, BLK // 128, 128), lambda i: (0, i, 0, 0)),
            pl.BlockSpec((BLK, H), lambda i: (i, 0)),
        ],
        out_specs=[
            pl.BlockSpec((BLK, H), lambda i: (i, 0)),
            pl.BlockSpec((BLK, H), lambda i: (i, 0)),
        ],
        out_shape=[
            jax.ShapeDtypeStruct((NP, H), jnp.float32),
            jax.ShapeDtypeStruct((NP, H), jnp.float32),
        ],
    )(degp, xw1)


def _mid_body(p_ref, y_ref, dinv_ref, b_ref, w_ref, o_ref):
    h = jnp.maximum(
        (p_ref[0] + p_ref[1] + y_ref[...]) * dinv_ref[...] + b_ref[...], 0.0)
    o_ref[...] = jnp.dot(h, w_ref[...],
                         preferred_element_type=jnp.float32,
                         precision=_HIGH) * dinv_ref[...]


def _tc_mid(p1, y1, dinvb, b1r, w2):
    return pl.pallas_call(
        _mid_body,
        grid=(NBLK,),
        in_specs=[
            pl.BlockSpec((2, BLK, H), lambda i: (0, i, 0)),
            pl.BlockSpec((BLK, H), lambda i: (i, 0)),
            pl.BlockSpec((BLK, H), lambda i: (i, 0)),
            pl.BlockSpec((1, H), lambda i: (0, 0)),
            pl.BlockSpec((H, H), lambda i: (0, 0)),
        ],
        out_specs=pl.BlockSpec((BLK, H), lambda i: (i, 0)),
        out_shape=jax.ShapeDtypeStruct((NP, H), jnp.float32),
    )(p1, y1, dinvb, b1r, w2)


def _final_body(p_ref, y_ref, dinv_ref, b_ref, batch_ref, wl_ref, bl_ref,
                o_ref, acc, cnt):
    i = pl.program_id(0)
    h = jnp.maximum(
        (p_ref[0] + p_ref[1] + y_ref[...]) * dinv_ref[...] + b_ref[...], 0.0)
    bid = batch_ref[0]
    gids = lax.broadcasted_iota(jnp.int32, (G, BLK), 0)
    maskT = (jnp.broadcast_to(bid, (G, BLK)) == gids).astype(jnp.float32)
    contrib = lax.dot_general(maskT, h, (((1,), (0,)), ((), ())),
                              preferred_element_type=jnp.float32,
                              precision=_HIGH)
    ones = jnp.ones((BLK, H), jnp.float32)
    ccnt = lax.dot_general(maskT, ones, (((1,), (0,)), ((), ())),
                           preferred_element_type=jnp.float32,
                           precision=_HIGH)

    @pl.when(i == 0)
    def _():
        acc[...] = contrib
        cnt[...] = ccnt

    @pl.when(i > 0)
    def _():
        acc[...] = acc[...] + contrib
        cnt[...] = cnt[...] + ccnt

    @pl.when(i == NBLK - 1)
    def _():
        g = acc[...] / jnp.maximum(cnt[...], 1.0)
        o_ref[...] = jnp.dot(g, wl_ref[...],
                             preferred_element_type=jnp.float32,
                             precision=_HIGH) + bl_ref[...]


def _tc_final(p2, y2, dinvb, b2r, batchr, wlinp, blinp):
    return pl.pallas_call(
        _final_body,
        grid=(NBLK,),
        in_specs=[
            pl.BlockSpec((2, BLK, H), lambda i: (0, i, 0)),
            pl.BlockSpec((BLK, H), lambda i: (i, 0)),
            pl.BlockSpec((BLK, H), lambda i: (i, 0)),
            pl.BlockSpec((1, H), lambda i: (0, 0)),
            pl.BlockSpec((1, 1, BLK), lambda i: (i, 0, 0)),
            pl.BlockSpec((H, OUTP), lambda i: (0, 0)),
            pl.BlockSpec((1, OUTP), lambda i: (0, 0)),
        ],
        out_specs=pl.BlockSpec((G, OUTP), lambda i: (0, 0)),
        out_shape=jax.ShapeDtypeStruct((G, OUTP), jnp.float32),
        scratch_shapes=[
            pltpu.VMEM((G, H), jnp.float32),
            pltpu.VMEM((G, H), jnp.float32),
        ],
    )(p2, y2, dinvb, b2r, batchr, wlinp, blinp)


def kernel(x, edge_index, batch, W1, b1, W2, b2, Wlin, blin):
    E = edge_index.shape[1]
    pad_e = EP - E
    pad_idx = N + jnp.arange(pad_e, dtype=jnp.int32) % (NP - N)
    src2d = jnp.concatenate([edge_index[0], pad_idx]).reshape(NW * NCH, CH)
    dst2d = jnp.concatenate([edge_index[1], pad_idx]).reshape(NW * NCH, CH)
    xp = jnp.zeros((NP, DIN), jnp.float32).at[:N].set(x)
    batchr = jnp.concatenate(
        [batch, jnp.full((NP - N,), G, jnp.int32)]).reshape(NBLK, 1, BLK)
    b1r = b1.reshape(1, H)
    b2r = b2.reshape(1, H)
    wlinp = jnp.zeros((H, OUTP), jnp.float32).at[:, :OUT].set(Wlin)
    blinp = jnp.zeros((1, OUTP), jnp.float32).at[0, :OUT].set(blin)

    zero128 = jnp.zeros((CH, H), jnp.float32)
    zflat = jnp.zeros((NP,), jnp.float32)

    degp = _sc_degree(dst2d, zflat).reshape(NW, NBLK, BLK // 128, 128)
    xw1 = _tc_matmul(xp, W1)
    y1, dinvb = _tc_first(degp, xw1)
    p1 = _sc_propagate(y1, src2d, dst2d, zero128)
    y2 = _tc_mid(p1, y1, dinvb, b1r, W2)
    p2 = _sc_propagate(y2, src2d, dst2d, zero128)
    logits_p = _tc_final(p2, y2, dinvb, b2r, batchr, wlinp, blinp)
    return logits_p[:, :OUT]


# speedup vs baseline: 3.0272x; 1.1102x over previous
import functools

import jax
import jax.numpy as jnp
from jax import lax
from jax.experimental import pallas as pl
from jax.experimental.pallas import tpu as pltpu
from jax.experimental.pallas import tpu_sc as plsc

N = 10000
DIN = 128
H = 128
OUT = 138
G = 64

NP = 10240
BLK = 512
NBLK = NP // BLK
OUTP = 256

NW = 32
CH = 128
NCH = 80
EP = NW * NCH * CH
STRIPE = NP // 16

NCHT = 2 * NCH
SPLIT0 = 80
SPLIT1 = NCHT - SPLIT0
PH = 16
IR = PH

_HIGH = lax.Precision.HIGHEST


@functools.cache
def _build_sc_degree():
    mesh = plsc.VectorSubcoreMesh(core_axis_name="c", subcore_axis_name="s")

    @functools.partial(
        pl.kernel,
        out_type=jax.ShapeDtypeStruct((NW * NP,), jnp.float32),
        mesh=mesh,
        scratch_types=[
            pltpu.VMEM((NP,), jnp.float32),
            pltpu.VMEM((NCH, CH), jnp.int32),
        ],
        compiler_params=pltpu.CompilerParams(needs_layout_passes=False),
    )
    def deg_kernel(dst_hbm, zero_hbm, out_hbm, hist, dstv):
        cid = lax.axis_index("c")
        sid = lax.axis_index("s")
        wid = sid * 2 + cid
        pltpu.sync_copy(zero_hbm, hist)
        pltpu.sync_copy(dst_hbm.at[pl.ds(wid * NCH, NCH)], dstv)
        ones16 = jnp.ones((16,), jnp.float32)

        @pl.loop(0, NCH)
        def _(j):
            for k in range(8):
                idx = dstv[j, pl.ds(k * 16, 16)]
                plsc.addupdate_scatter(hist, [idx], ones16)

        pltpu.sync_copy(hist, out_hbm.at[pl.ds(wid * NP, NP)])

    return deg_kernel


def _sc_degree(dst2d, zflat):
    return _build_sc_degree()(dst2d, zflat)


@functools.cache
def _build_sc_propagate():
    mesh = plsc.VectorSubcoreMesh(core_axis_name="c", subcore_axis_name="s")

    @functools.partial(
        pl.kernel,
        out_type=jax.ShapeDtypeStruct((2, NP, H), jnp.float32),
        mesh=mesh,
        scratch_types=[
            pltpu.VMEM_SHARED((NP, H), jnp.float32),
            pltpu.VMEM((IR, CH), jnp.int32),
            pltpu.VMEM((IR, CH), jnp.int32),
            pltpu.VMEM((CH, H), jnp.float32),
            pltpu.VMEM((CH, H), jnp.float32),
            pltpu.SemaphoreType.DMA,
            pltpu.SemaphoreType.DMA,
        ],
    )
    def prop_kernel(y_hbm, src_hbm, dst_hbm, zero_hbm, out_hbm,
                    acc, srcv, dstv, ra, rb, sa, sb):
        cid = lax.axis_index("c")
        sid = lax.axis_index("s")

        pltpu.sync_copy(zero_hbm, ra)
        for k in range(STRIPE // CH):
            pltpu.sync_copy(ra, acc.at[pl.ds(sid * STRIPE + k * CH, CH)])

        plsc.subcore_barrier()

        def run_edges(base_row, nch):
            hn = PH
            for p in range(nch // PH):
                row0 = base_row + p * hn
                pltpu.sync_copy(src_hbm.at[pl.ds(row0, hn)],
                                srcv.at[pl.ds(0, hn)])
                pltpu.sync_copy(dst_hbm.at[pl.ds(row0, hn)],
                                dstv.at[pl.ds(0, hn)])

                pltpu.async_copy(y_hbm.at[srcv.at[0]], ra, sa)

                @pl.loop(0, hn, step=2)
                def _(j):
                    pltpu.async_copy(y_hbm.at[srcv.at[j + 1]], rb, sb)
                    pltpu.make_async_copy(y_hbm.at[srcv.at[j]], ra, sa).wait()
                    pltpu.sync_copy(ra, acc.at[dstv.at[j]], add=True)

                    @pl.when(j + 2 < hn)
                    def _():
                        pltpu.async_copy(y_hbm.at[srcv.at[j + 2]], ra, sa)

                    pltpu.make_async_copy(y_hbm.at[srcv.at[j + 1]], rb, sb).wait()
                    pltpu.sync_copy(rb, acc.at[dstv.at[j + 1]], add=True)

        @pl.when(cid == 0)
        def _():
            run_edges(sid * SPLIT0, SPLIT0)

        @pl.when(cid == 1)
        def _():
            run_edges(16 * SPLIT0 + sid * SPLIT1, SPLIT1)

        plsc.subcore_barrier()
        pltpu.sync_copy(acc.at[pl.ds(sid * STRIPE, STRIPE)],
                        out_hbm.at[cid, pl.ds(sid * STRIPE, STRIPE)])

    return prop_kernel


def _sc_propagate(y, src2d, dst2d, zero128):
    return _build_sc_propagate()(y, src2d, dst2d, zero128)


def _first_body(deg_ref, x_ref, w_ref, y_ref, dinv_ref):
    s = jnp.sum(deg_ref[:, 0], axis=0)
    dinv = lax.rsqrt(s + 1.0)
    r = lax.broadcasted_iota(jnp.int32, (BLK, H), 0)
    c = lax.broadcasted_iota(jnp.int32, (BLK, H), 1)
    esel = (lax.broadcasted_iota(jnp.int32, (BLK, BLK // 128), 1)
            == r[:, :BLK // 128] // 128).astype(jnp.float32)
    expand = lax.dot_general(esel, dinv, (((1,), (0,)), ((), ())),
                             preferred_element_type=jnp.float32,
                             precision=_HIGH)
    onehot = jnp.where(c == r % 128, expand, 0.0)
    dinvb = lax.dot_general(onehot, jnp.ones((H, H), jnp.float32),
                            (((1,), (0,)), ((), ())),
                            preferred_element_type=jnp.float32,
                            precision=_HIGH)
    xw = jnp.dot(x_ref[...], w_ref[...],
                 preferred_element_type=jnp.float32, precision=_HIGH)
    y_ref[...] = xw * dinvb
    dinv_ref[...] = dinvb


def _tc_first(degp, xp, w1):
    return pl.pallas_call(
        _first_body,
        grid=(NBLK,),
        in_specs=[
            pl.BlockSpec((NW, 1, BLK // 128, 128), lambda i: (0, i, 0, 0)),
            pl.BlockSpec((BLK, DIN), lambda i: (i, 0)),
            pl.BlockSpec((DIN, H), lambda i: (0, 0)),
        ],
        out_specs=[
            pl.BlockSpec((BLK, H), lambda i: (i, 0)),
            pl.BlockSpec((BLK, H), lambda i: (i, 0)),
        ],
        out_shape=[
            jax.ShapeDtypeStruct((NP, H), jnp.float32),
            jax.ShapeDtypeStruct((NP, H), jnp.float32),
        ],
    )(degp, xp, w1)


def _mid_body(p_ref, y_ref, dinv_ref, b_ref, w_ref, o_ref):
    h = jnp.maximum(
        (p_ref[0] + p_ref[1] + y_ref[...]) * dinv_ref[...] + b_ref[...], 0.0)
    o_ref[...] = jnp.dot(h, w_ref[...],
                         preferred_element_type=jnp.float32,
                         precision=_HIGH) * dinv_ref[...]


def _tc_mid(p1, y1, dinvb, b1r, w2):
    return pl.pallas_call(
        _mid_body,
        grid=(NBLK,),
        in_specs=[
            pl.BlockSpec((2, BLK, H), lambda i: (0, i, 0)),
            pl.BlockSpec((BLK, H), lambda i: (i, 0)),
            pl.BlockSpec((BLK, H), lambda i: (i, 0)),
            pl.BlockSpec((1, H), lambda i: (0, 0)),
            pl.BlockSpec((H, H), lambda i: (0, 0)),
        ],
        out_specs=pl.BlockSpec((BLK, H), lambda i: (i, 0)),
        out_shape=jax.ShapeDtypeStruct((NP, H), jnp.float32),
    )(p1, y1, dinvb, b1r, w2)


def _final_body(p_ref, y_ref, dinv_ref, b_ref, batch_ref, wl_ref, bl_ref,
                o_ref, acc, cnt):
    i = pl.program_id(0)
    h = jnp.maximum(
        (p_ref[0] + p_ref[1] + y_ref[...]) * dinv_ref[...] + b_ref[...], 0.0)
    bid = batch_ref[0]
    gids = lax.broadcasted_iota(jnp.int32, (G, BLK), 0)
    maskT = (jnp.broadcast_to(bid, (G, BLK)) == gids).astype(jnp.float32)
    contrib = lax.dot_general(maskT, h, (((1,), (0,)), ((), ())),
                              preferred_element_type=jnp.float32,
                              precision=_HIGH)
    ones = jnp.ones((BLK, H), jnp.float32)
    ccnt = lax.dot_general(maskT, ones, (((1,), (0,)), ((), ())),
                           preferred_element_type=jnp.float32,
                           precision=_HIGH)

    @pl.when(i == 0)
    def _():
        acc[...] = contrib
        cnt[...] = ccnt

    @pl.when(i > 0)
    def _():
        acc[...] = acc[...] + contrib
        cnt[...] = cnt[...] + ccnt

    @pl.when(i == NBLK - 1)
    def _():
        g = acc[...] / jnp.maximum(cnt[...], 1.0)
        o_ref[...] = jnp.dot(g, wl_ref[...],
                             preferred_element_type=jnp.float32,
                             precision=_HIGH) + bl_ref[...]


def _tc_final(p2, y2, dinvb, b2r, batchr, wlinp, blinp):
    return pl.pallas_call(
        _final_body,
        grid=(NBLK,),
        in_specs=[
            pl.BlockSpec((2, BLK, H), lambda i: (0, i, 0)),
            pl.BlockSpec((BLK, H), lambda i: (i, 0)),
            pl.BlockSpec((BLK, H), lambda i: (i, 0)),
            pl.BlockSpec((1, H), lambda i: (0, 0)),
            pl.BlockSpec((1, 1, BLK), lambda i: (i, 0, 0)),
            pl.BlockSpec((H, OUTP), lambda i: (0, 0)),
            pl.BlockSpec((1, OUTP), lambda i: (0, 0)),
        ],
        out_specs=pl.BlockSpec((G, OUTP), lambda i: (0, 0)),
        out_shape=jax.ShapeDtypeStruct((G, OUTP), jnp.float32),
        scratch_shapes=[
            pltpu.VMEM((G, H), jnp.float32),
            pltpu.VMEM((G, H), jnp.float32),
        ],
    )(p2, y2, dinvb, b2r, batchr, wlinp, blinp)


def kernel(x, edge_index, batch, W1, b1, W2, b2, Wlin, blin):
    E = edge_index.shape[1]
    pad_e = EP - E
    pad_idx = N + jnp.arange(pad_e, dtype=jnp.int32) % (NP - N)
    src2d = jnp.concatenate([edge_index[0], pad_idx]).reshape(NW * NCH, CH)
    dst2d = jnp.concatenate([edge_index[1], pad_idx]).reshape(NW * NCH, CH)
    xp = jnp.zeros((NP, DIN), jnp.float32).at[:N].set(x)
    batchr = jnp.concatenate(
        [batch, jnp.full((NP - N,), G, jnp.int32)]).reshape(NBLK, 1, BLK)
    b1r = b1.reshape(1, H)
    b2r = b2.reshape(1, H)
    wlinp = jnp.zeros((H, OUTP), jnp.float32).at[:, :OUT].set(Wlin)
    blinp = jnp.zeros((1, OUTP), jnp.float32).at[0, :OUT].set(blin)

    zero128 = jnp.zeros((CH, H), jnp.float32)
    zflat = jnp.zeros((NP,), jnp.float32)

    degp = _sc_degree(dst2d, zflat).reshape(NW, NBLK, BLK // 128, 128)
    y1, dinvb = _tc_first(degp, xp, W1)
    p1 = _sc_propagate(y1, src2d, dst2d, zero128)
    y2 = _tc_mid(p1, y1, dinvb, b1r, W2)
    p2 = _sc_propagate(y2, src2d, dst2d, zero128)
    logits_p = _tc_final(p2, y2, dinvb, b2r, batchr, wlinp, blinp)
    return logits_p[:, :OUT]


# speedup vs baseline: 3.1211x; 1.0310x over previous
import functools

import jax
import jax.numpy as jnp
from jax import lax
from jax.experimental import pallas as pl
from jax.experimental.pallas import tpu as pltpu
from jax.experimental.pallas import tpu_sc as plsc

N = 10000
DIN = 128
H = 128
OUT = 138
G = 64

NP = 10240
BLK = 512
NBLK = NP // BLK
OUTP = 256

NW = 32
CH = 128
NCH = 80
EP = NW * NCH * CH
STRIPE = NP // 16

NCHT = 2 * NCH
SPLIT0 = 80
SPLIT1 = NCHT - SPLIT0
PH = 16
IR = PH

_HIGH = lax.Precision.HIGHEST


@functools.cache
def _build_sc_degree():
    mesh = plsc.VectorSubcoreMesh(core_axis_name="c", subcore_axis_name="s")

    @functools.partial(
        pl.kernel,
        out_type=jax.ShapeDtypeStruct((NW * NP,), jnp.float32),
        mesh=mesh,
        scratch_types=[
            pltpu.VMEM((NP,), jnp.float32),
            pltpu.VMEM((NCH, CH), jnp.int32),
        ],
        compiler_params=pltpu.CompilerParams(needs_layout_passes=False),
    )
    def deg_kernel(dst_hbm, zero_hbm, out_hbm, hist, dstv):
        cid = lax.axis_index("c")
        sid = lax.axis_index("s")
        wid = sid * 2 + cid
        pltpu.sync_copy(zero_hbm, hist)
        pltpu.sync_copy(dst_hbm.at[pl.ds(wid * NCH, NCH)], dstv)
        ones16 = jnp.ones((16,), jnp.float32)

        @pl.loop(0, NCH)
        def _(j):
            for k in range(8):
                idx = dstv[j, pl.ds(k * 16, 16)]
                plsc.addupdate_scatter(hist, [idx], ones16)

        pltpu.sync_copy(hist, out_hbm.at[pl.ds(wid * NP, NP)])

    return deg_kernel


def _sc_degree(dst2d, zflat):
    return _build_sc_degree()(dst2d, zflat)


@functools.cache
def _build_sc_propagate():
    mesh = plsc.VectorSubcoreMesh(core_axis_name="c", subcore_axis_name="s")

    @functools.partial(
        pl.kernel,
        out_type=jax.ShapeDtypeStruct((2, NP, H), jnp.float32),
        mesh=mesh,
        scratch_types=[
            pltpu.VMEM_SHARED((NP, H), jnp.float32),
            pltpu.VMEM((IR, CH), jnp.int32),
            pltpu.VMEM((IR, CH), jnp.int32),
            pltpu.VMEM((IR, CH), jnp.int32),
            pltpu.VMEM((IR, CH), jnp.int32),
            pltpu.VMEM((CH, H), jnp.float32),
            pltpu.VMEM((CH, H), jnp.float32),
            pltpu.SemaphoreType.DMA,
            pltpu.SemaphoreType.DMA,
            pltpu.SemaphoreType.DMA,
            pltpu.SemaphoreType.DMA,
        ],
    )
    def prop_kernel(y_hbm, src_hbm, dst_hbm, zero_hbm, out_hbm,
                    acc, srcv, dstv, srcw, dstw, ra, rb, sa, sb, si, sj):
        cid = lax.axis_index("c")
        sid = lax.axis_index("s")

        pltpu.sync_copy(zero_hbm, ra)
        for k in range(STRIPE // CH):
            pltpu.sync_copy(ra, acc.at[pl.ds(sid * STRIPE + k * CH, CH)])

        plsc.subcore_barrier()

        def run_edges(base_row, nch):
            hn = PH
            nphase = nch // PH

            def idx_copies(p):
                row0 = base_row + p * hn
                sv, dv, sem = (srcv, dstv, si) if p % 2 == 0 else (srcw, dstw, sj)
                return (sv, dv,
                        pltpu.make_async_copy(src_hbm.at[pl.ds(row0, hn)], sv, sem),
                        pltpu.make_async_copy(dst_hbm.at[pl.ds(row0, hn)], dv, sem))

            _, _, c0, c1 = idx_copies(0)
            c0.start()
            c1.start()
            for p in range(nphase):
                sv, dv, ca, cb = idx_copies(p)
                ca.wait()
                cb.wait()
                if p + 1 < nphase:
                    _, _, na, nb = idx_copies(p + 1)
                    na.start()
                    nb.start()

                pltpu.async_copy(y_hbm.at[sv.at[0]], ra, sa)

                @pl.loop(0, hn, step=2)
                def _(j):
                    pltpu.async_copy(y_hbm.at[sv.at[j + 1]], rb, sb)
                    pltpu.make_async_copy(y_hbm.at[sv.at[j]], ra, sa).wait()
                    pltpu.sync_copy(ra, acc.at[dv.at[j]], add=True)

                    @pl.when(j + 2 < hn)
                    def _():
                        pltpu.async_copy(y_hbm.at[sv.at[j + 2]], ra, sa)

                    pltpu.make_async_copy(y_hbm.at[sv.at[j + 1]], rb, sb).wait()
                    pltpu.sync_copy(rb, acc.at[dv.at[j + 1]], add=True)

        @pl.when(cid == 0)
        def _():
            run_edges(sid * SPLIT0, SPLIT0)

        @pl.when(cid == 1)
        def _():
            run_edges(16 * SPLIT0 + sid * SPLIT1, SPLIT1)

        plsc.subcore_barrier()
        pltpu.sync_copy(acc.at[pl.ds(sid * STRIPE, STRIPE)],
                        out_hbm.at[cid, pl.ds(sid * STRIPE, STRIPE)])

    return prop_kernel


def _sc_propagate(y, src2d, dst2d, zero128):
    return _build_sc_propagate()(y, src2d, dst2d, zero128)


def _first_body(deg_ref, x_ref, w_ref, y_ref, dinv_ref):
    s = jnp.sum(deg_ref[:, 0], axis=0)
    dinv = lax.rsqrt(s + 1.0)
    r = lax.broadcasted_iota(jnp.int32, (BLK, H), 0)
    c = lax.broadcasted_iota(jnp.int32, (BLK, H), 1)
    esel = (lax.broadcasted_iota(jnp.int32, (BLK, BLK // 128), 1)
            == r[:, :BLK // 128] // 128).astype(jnp.float32)
    expand = lax.dot_general(esel, dinv, (((1,), (0,)), ((), ())),
                             preferred_element_type=jnp.float32,
                             precision=_HIGH)
    onehot = jnp.where(c == r % 128, expand, 0.0)
    dinvb = lax.dot_general(onehot, jnp.ones((H, H), jnp.float32),
                            (((1,), (0,)), ((), ())),
                            preferred_element_type=jnp.float32,
                            precision=_HIGH)
    xw = jnp.dot(x_ref[...], w_ref[...],
                 preferred_element_type=jnp.float32, precision=_HIGH)
    y_ref[...] = xw * dinvb
    dinv_ref[...] = dinvb


def _tc_first(degp, xp, w1):
    return pl.pallas_call(
        _first_body,
        grid=(NBLK,),
        in_specs=[
            pl.BlockSpec((NW, 1, BLK // 128, 128), lambda i: (0, i, 0, 0)),
            pl.BlockSpec((BLK, DIN), lambda i: (i, 0)),
            pl.BlockSpec((DIN, H), lambda i: (0, 0)),
        ],
        out_specs=[
            pl.BlockSpec((BLK, H), lambda i: (i, 0)),
            pl.BlockSpec((BLK, H), lambda i: (i, 0)),
        ],
        out_shape=[
            jax.ShapeDtypeStruct((NP, H), jnp.float32),
            jax.ShapeDtypeStruct((NP, H), jnp.float32),
        ],
    )(degp, xp, w1)


def _mid_body(p_ref, y_ref, dinv_ref, b_ref, w_ref, o_ref):
    h = jnp.maximum(
        (p_ref[0] + p_ref[1] + y_ref[...]) * dinv_ref[...] + b_ref[...], 0.0)
    o_ref[...] = jnp.dot(h, w_ref[...],
                         preferred_element_type=jnp.float32,
                         precision=_HIGH) * dinv_ref[...]


def _tc_mid(p1, y1, dinvb, b1r, w2):
    return pl.pallas_call(
        _mid_body,
        grid=(NBLK,),
        in_specs=[
            pl.BlockSpec((2, BLK, H), lambda i: (0, i, 0)),
            pl.BlockSpec((BLK, H), lambda i: (i, 0)),
            pl.BlockSpec((BLK, H), lambda i: (i, 0)),
            pl.BlockSpec((1, H), lambda i: (0, 0)),
            pl.BlockSpec((H, H), lambda i: (0, 0)),
        ],
        out_specs=pl.BlockSpec((BLK, H), lambda i: (i, 0)),
        out_shape=jax.ShapeDtypeStruct((NP, H), jnp.float32),
    )(p1, y1, dinvb, b1r, w2)


def _final_body(p_ref, y_ref, dinv_ref, b_ref, batch_ref, wl_ref, bl_ref,
                o_ref, acc, cnt):
    i = pl.program_id(0)
    h = jnp.maximum(
        (p_ref[0] + p_ref[1] + y_ref[...]) * dinv_ref[...] + b_ref[...], 0.0)
    bid = batch_ref[0]
    gids = lax.broadcasted_iota(jnp.int32, (G, BLK), 0)
    maskT = (jnp.broadcast_to(bid, (G, BLK)) == gids).astype(jnp.float32)
    contrib = lax.dot_general(maskT, h, (((1,), (0,)), ((), ())),
                              preferred_element_type=jnp.float32,
                              precision=_HIGH)
    ones = jnp.ones((BLK, H), jnp.float32)
    ccnt = lax.dot_general(maskT, ones, (((1,), (0,)), ((), ())),
                           preferred_element_type=jnp.float32,
                           precision=_HIGH)

    @pl.when(i == 0)
    def _():
        acc[...] = contrib
        cnt[...] = ccnt

    @pl.when(i > 0)
    def _():
        acc[...] = acc[...] + contrib
        cnt[...] = cnt[...] + ccnt

    @pl.when(i == NBLK - 1)
    def _():
        g = acc[...] / jnp.maximum(cnt[...], 1.0)
        o_ref[...] = jnp.dot(g, wl_ref[...],
                             preferred_element_type=jnp.float32,
                             precision=_HIGH) + bl_ref[...]


def _tc_final(p2, y2, dinvb, b2r, batchr, wlinp, blinp):
    return pl.pallas_call(
        _final_body,
        grid=(NBLK,),
        in_specs=[
            pl.BlockSpec((2, BLK, H), lambda i: (0, i, 0)),
            pl.BlockSpec((BLK, H), lambda i: (i, 0)),
            pl.BlockSpec((BLK, H), lambda i: (i, 0)),
            pl.BlockSpec((1, H), lambda i: (0, 0)),
            pl.BlockSpec((1, 1, BLK), lambda i: (i, 0, 0)),
            pl.BlockSpec((H, OUTP), lambda i: (0, 0)),
            pl.BlockSpec((1, OUTP), lambda i: (0, 0)),
        ],
        out_specs=pl.BlockSpec((G, OUTP), lambda i: (0, 0)),
        out_shape=jax.ShapeDtypeStruct((G, OUTP), jnp.float32),
        scratch_shapes=[
            pltpu.VMEM((G, H), jnp.float32),
            pltpu.VMEM((G, H), jnp.float32),
        ],
    )(p2, y2, dinvb, b2r, batchr, wlinp, blinp)


def kernel(x, edge_index, batch, W1, b1, W2, b2, Wlin, blin):
    E = edge_index.shape[1]
    pad_e = EP - E
    pad_idx = N + jnp.arange(pad_e, dtype=jnp.int32) % (NP - N)
    src2d = jnp.concatenate([edge_index[0], pad_idx]).reshape(NW * NCH, CH)
    dst2d = jnp.concatenate([edge_index[1], pad_idx]).reshape(NW * NCH, CH)
    xp = jnp.zeros((NP, DIN), jnp.float32).at[:N].set(x)
    batchr = jnp.concatenate(
        [batch, jnp.full((NP - N,), G, jnp.int32)]).reshape(NBLK, 1, BLK)
    b1r = b1.reshape(1, H)
    b2r = b2.reshape(1, H)
    wlinp = jnp.zeros((H, OUTP), jnp.float32).at[:, :OUT].set(Wlin)
    blinp = jnp.zeros((1, OUTP), jnp.float32).at[0, :OUT].set(blin)

    zero128 = jnp.zeros((CH, H), jnp.float32)
    zflat = jnp.zeros((NP,), jnp.float32)

    degp = _sc_degree(dst2d, zflat).reshape(NW, NBLK, BLK // 128, 128)
    y1, dinvb = _tc_first(degp, xp, W1)
    p1 = _sc_propagate(y1, src2d, dst2d, zero128)
    y2 = _tc_mid(p1, y1, dinvb, b1r, W2)
    p2 = _sc_propagate(y2, src2d, dst2d, zero128)
    logits_p = _tc_final(p2, y2, dinvb, b2r, batchr, wlinp, blinp)
    return logits_p[:, :OUT]
